# Initial kernel scaffold; baseline (speedup 1.0000x reference)
import jax
import jax.numpy as jnp
from jax.experimental import pallas as pl


def kernel(inp, values, mask, weight):
    raise NotImplementedError("write your pallas kernel here")


# speedup vs baseline: 10.8605x; 10.8605x over previous
import functools

import jax
import jax.numpy as jnp
from jax import lax
from jax.experimental import pallas as pl
from jax.experimental.pallas import tpu as pltpu
from jax.experimental.pallas import tpu_sc as plsc

B, L, H = 1024, 200, 128
V = 100000

NC, NS, LANES = 2, 16, 16
NW = NC * NS
BL = B * L
PER_W = BL // NW


def _sc_gather(table, idx_flat):
    mesh = plsc.VectorSubcoreMesh(core_axis_name="c", subcore_axis_name="s")

    @functools.partial(
        pl.kernel,
        mesh=mesh,
        out_type=jax.ShapeDtypeStruct((BL,), jnp.float32),
        scratch_types=[
            pltpu.VMEM((V,), jnp.float32),
            pltpu.VMEM((PER_W,), jnp.int32),
            pltpu.VMEM((PER_W,), jnp.float32),
        ],
        compiler_params=pltpu.CompilerParams(needs_layout_passes=False),
    )
    def gather_kernel(table_hbm, idx_hbm, out_hbm, table_v, idx_v, out_v):
        wid = lax.axis_index("s") * NC + lax.axis_index("c")
        base = wid * PER_W
        pltpu.sync_copy(table_hbm, table_v)
        pltpu.sync_copy(idx_hbm.at[pl.ds(base, PER_W)], idx_v)

        def body(j, carry):
            sl = pl.ds(j * LANES, LANES)
            out_v[sl] = plsc.load_gather(table_v, [idx_v[sl]])
            return carry

        lax.fori_loop(0, PER_W // LANES, body, 0)
        pltpu.sync_copy(out_v, out_hbm.at[pl.ds(base, PER_W)])

    return gather_kernel(table, idx_flat)


BT = 8


def _pool_body(w_ref, mask_ref, v_ref, out_ref):
    w = w_ref[...]
    m = mask_ref[...]
    wm = jnp.where((1.0 - m) > 0.5, -jnp.inf, w)
    mx = jnp.max(wm, axis=1, keepdims=True)
    e = jnp.exp(wm - mx)
    s = jnp.sum(e, axis=1, keepdims=True)
    p = (e / s) * m
    v = v_ref[...]
    out_ref[...] = jnp.sum(p[:, :, None] * v, axis=1)


def kernel(inp, values, mask, weight):
    w_flat = _sc_gather(weight.reshape(V), inp.reshape(BL))
    w = w_flat.reshape(B, L)
    out = pl.pallas_call(
        _pool_body,
        grid=(B // BT,),
        in_specs=[
            pl.BlockSpec((BT, L), lambda i: (i, 0)),
            pl.BlockSpec((BT, L), lambda i: (i, 0)),
            pl.BlockSpec((BT, L, H), lambda i: (i, 0, 0)),
        ],
        out_specs=pl.BlockSpec((BT, H), lambda i: (i, 0)),
        out_shape=jax.ShapeDtypeStruct((B, H), jnp.float32),
    )(w, mask, values)
    return out


# speedup vs baseline: 20.7305x; 1.9088x over previous
import functools

import jax
import jax.numpy as jnp
from jax import lax
from jax.experimental import pallas as pl
from jax.experimental.pallas import tpu as pltpu
from jax.experimental.pallas import tpu_sc as plsc

B, L, H = 1024, 200, 128
V = 100000

NC, NS, LANES = 2, 16, 16
NW = NC * NS
BL = B * L
PER_W = BL // NW


def _sc_gather(table, idx_flat):
    mesh = plsc.VectorSubcoreMesh(core_axis_name="c", subcore_axis_name="s")

    @functools.partial(
        pl.kernel,
        mesh=mesh,
        out_type=jax.ShapeDtypeStruct((BL,), jnp.float32),
        scratch_types=[
            pltpu.VMEM((V,), jnp.float32),
            pltpu.VMEM((PER_W,), jnp.int32),
            pltpu.VMEM((PER_W,), jnp.float32),
        ],
        compiler_params=pltpu.CompilerParams(needs_layout_passes=False),
    )
    def gather_kernel(table_hbm, idx_hbm, out_hbm, table_v, idx_v, out_v):
        wid = lax.axis_index("s") * NC + lax.axis_index("c")
        base = wid * PER_W
        pltpu.sync_copy(table_hbm, table_v)
        pltpu.sync_copy(idx_hbm.at[pl.ds(base, PER_W)], idx_v)

        def body(j, carry):
            sl = pl.ds(j * LANES, LANES)
            out_v[sl] = plsc.load_gather(table_v, [idx_v[sl]])
            return carry

        lax.fori_loop(0, PER_W // LANES, body, 0)
        pltpu.sync_copy(out_v, out_hbm.at[pl.ds(base, PER_W)])

    return gather_kernel(table, idx_flat)


BT = 64


def _pool_body(w_ref, mask_ref, v_ref, out_ref):
    w = w_ref[...]
    m = mask_ref[...]
    wm = jnp.where((1.0 - m) > 0.5, -jnp.inf, w)
    mx = jnp.max(wm, axis=1, keepdims=True)
    e = jnp.exp(wm - mx)
    s = jnp.sum(e, axis=1, keepdims=True)
    p = (e / s) * m
    v = v_ref[...]
    out_ref[...] = jax.lax.dot_general(
        p, v, (((1,), (1,)), ((0,), (0,))),
        preferred_element_type=jnp.float32,
    )


def kernel(inp, values, mask, weight):
    w_flat = _sc_gather(weight.reshape(V), inp.reshape(BL))
    w = w_flat.reshape(B, L)
    out = pl.pallas_call(
        _pool_body,
        grid=(B // BT,),
        in_specs=[
            pl.BlockSpec((BT, L), lambda i: (i, 0)),
            pl.BlockSpec((BT, L), lambda i: (i, 0)),
            pl.BlockSpec((BT, L, H), lambda i: (i, 0, 0)),
        ],
        out_specs=pl.BlockSpec((BT, H), lambda i: (i, 0)),
        out_shape=jax.ShapeDtypeStruct((B, H), jnp.float32),
    )(w, mask, values)
    return out


# speedup vs baseline: 23.3738x; 1.1275x over previous
import functools

import jax
import jax.numpy as jnp
from jax import lax
from jax.experimental import pallas as pl
from jax.experimental.pallas import tpu as pltpu
from jax.experimental.pallas import tpu_sc as plsc

B, L, H = 1024, 200, 128
V = 100000

NC, NS, LANES = 2, 16, 16
NW = NC * NS
BL = B * L
PER_W = BL // NW


def _sc_gather(table, idx_flat):
    mesh = plsc.VectorSubcoreMesh(core_axis_name="c", subcore_axis_name="s")

    @functools.partial(
        pl.kernel,
        mesh=mesh,
        out_type=jax.ShapeDtypeStruct((BL,), jnp.float32),
        scratch_types=[
            pltpu.VMEM_SHARED((V,), jnp.float32),
            pltpu.VMEM((V,), jnp.float32),
            pltpu.VMEM((PER_W,), jnp.int32),
            pltpu.VMEM((PER_W,), jnp.float32),
            pltpu.SemaphoreType.DMA,
        ],
        compiler_params=pltpu.CompilerParams(needs_layout_passes=False),
    )
    def gather_kernel(table_hbm, idx_hbm, out_hbm, table_sh, table_v, idx_v,
                      out_v, sem):
        sid = lax.axis_index("s")
        wid = sid * NC + lax.axis_index("c")
        base = wid * PER_W
        idx_cp = pltpu.async_copy(idx_hbm.at[pl.ds(base, PER_W)], idx_v, sem)
        @pl.when(sid == 0)
        def _():
            pltpu.sync_copy(table_hbm, table_sh)
        plsc.subcore_barrier()
        pltpu.sync_copy(table_sh, table_v)
        idx_cp.wait()

        def body(j, carry):
            for u in range(4):
                sl = pl.ds((j * 4 + u) * LANES, LANES)
                out_v[sl] = plsc.load_gather(table_v, [idx_v[sl]])
            return carry

        lax.fori_loop(0, PER_W // (4 * LANES), body, 0)
        pltpu.sync_copy(out_v, out_hbm.at[pl.ds(base, PER_W)])

    return gather_kernel(table, idx_flat)


BT = 64


def _pool_body(w_ref, mask_ref, v_ref, out_ref):
    w = w_ref[...]
    m = mask_ref[...]
    wm = jnp.where((1.0 - m) > 0.5, -jnp.inf, w)
    mx = jnp.max(wm, axis=1, keepdims=True)
    e = jnp.exp(wm - mx)
    s = jnp.sum(e, axis=1, keepdims=True)
    p = (e / s) * m
    v = v_ref[...]
    out_ref[...] = jax.lax.dot_general(
        p, v, (((1,), (1,)), ((0,), (0,))),
        preferred_element_type=jnp.float32,
    )


def kernel(inp, values, mask, weight):
    w_flat = _sc_gather(weight.reshape(V), inp.reshape(BL))
    w = w_flat.reshape(B, L)
    out = pl.pallas_call(
        _pool_body,
        grid=(B // BT,),
        in_specs=[
            pl.BlockSpec((BT, L), lambda i: (i, 0)),
            pl.BlockSpec((BT, L), lambda i: (i, 0)),
            pl.BlockSpec((BT, L, H), lambda i: (i, 0, 0)),
        ],
        out_specs=pl.BlockSpec((BT, H), lambda i: (i, 0)),
        out_shape=jax.ShapeDtypeStruct((B, H), jnp.float32),
    )(w, mask, values)
    return out


# speedup vs baseline: 25.0882x; 1.0733x over previous
import functools

import jax
import jax.numpy as jnp
from jax import lax
from jax.experimental import pallas as pl
from jax.experimental.pallas import tpu as pltpu
from jax.experimental.pallas import tpu_sc as plsc

B, L, H = 1024, 200, 128
V = 100000

NC, NS, LANES = 2, 16, 16
NW = NC * NS
ROWS_W = B // NW
NVEC = L // LANES
TAIL = L - NVEC * LANES


def _sc_gather(table, idx2d):
    mesh = plsc.VectorSubcoreMesh(core_axis_name="c", subcore_axis_name="s")

    @functools.partial(
        pl.kernel,
        mesh=mesh,
        out_type=jax.ShapeDtypeStruct((B, L), jnp.float32),
        scratch_types=[
            pltpu.VMEM_SHARED((V,), jnp.float32),
            pltpu.VMEM((V,), jnp.float32),
            pltpu.VMEM((ROWS_W, L), jnp.int32),
            pltpu.VMEM((ROWS_W, L), jnp.float32),
            pltpu.SemaphoreType.DMA,
        ],
        compiler_params=pltpu.CompilerParams(needs_layout_passes=False),
    )
    def gather_kernel(table_hbm, idx_hbm, out_hbm, table_sh, table_v, idx_v,
                      out_v, sem):
        sid = lax.axis_index("s")
        wid = sid * NC + lax.axis_index("c")
        base = wid * ROWS_W
        idx_cp = pltpu.async_copy(idx_hbm.at[pl.ds(base, ROWS_W)], idx_v, sem)
        @pl.when(sid == 0)
        def _():
            pltpu.sync_copy(table_hbm, table_sh)
        plsc.subcore_barrier()
        pltpu.sync_copy(table_sh, table_v)
        idx_cp.wait()

        def row(r, carry):
            for u in range(NVEC):
                sl = pl.ds(u * LANES, LANES)
                out_v[r, sl] = plsc.load_gather(table_v, [idx_v[r, sl]])
            if TAIL:
                sl = pl.ds(L - LANES, LANES)
                out_v[r, sl] = plsc.load_gather(table_v, [idx_v[r, sl]])
            return carry

        lax.fori_loop(0, ROWS_W, row, 0)
        pltpu.sync_copy(out_v, out_hbm.at[pl.ds(base, ROWS_W)])

    return gather_kernel(table, idx2d)


BT = 64


def _pool_body(w_ref, mask_ref, v_ref, out_ref):
    w = w_ref[...]
    m = mask_ref[...]
    wm = jnp.where((1.0 - m) > 0.5, -jnp.inf, w)
    mx = jnp.max(wm, axis=1, keepdims=True)
    e = jnp.exp(wm - mx)
    s = jnp.sum(e, axis=1, keepdims=True)
    p = (e / s) * m
    v = v_ref[...]
    out_ref[...] = jax.lax.dot_general(
        p, v, (((1,), (1,)), ((0,), (0,))),
        preferred_element_type=jnp.float32,
    )


def kernel(inp, values, mask, weight):
    w = _sc_gather(weight.reshape(V), inp)
    out = pl.pallas_call(
        _pool_body,
        grid=(B // BT,),
        in_specs=[
            pl.BlockSpec((BT, L), lambda i: (i, 0)),
            pl.BlockSpec((BT, L), lambda i: (i, 0)),
            pl.BlockSpec((BT, L, H), lambda i: (i, 0, 0)),
        ],
        out_specs=pl.BlockSpec((BT, H), lambda i: (i, 0)),
        out_shape=jax.ShapeDtypeStruct((B, H), jnp.float32),
    )(w, mask, values)
    return out
